# baseline (speedup 1.0000x reference)
import jax
import jax.numpy as jnp
from jax import lax
from jax.experimental import pallas as pl
from jax.experimental.pallas import tpu as pltpu
from jax.experimental.pallas import tpu_sc as plsc

EMBED = 64
B = 16384
P = 20
M = 20
C = P + M
CPAD = 48

NC = 2
NS = 16
NW = NC * NS
RW = B // NW
CB = 32
NCHUNK = RW // CB
IDX_W = 128
NIDX = CB * C // IDX_W
LANES = 16


def _body(tgt_hbm, ctx_hbm, in_hbm, out_hbm, scores_hbm,
          tgt_idx, ctx_idx, v_rows, u_rows, out_v, sem):
    wid = lax.axis_index("s") * NC + lax.axis_index("c")
    lane = lax.broadcasted_iota(jnp.int32, (LANES,), 0)

    def chunk_body(ci, carry):
        b0 = pl.multiple_of(wid * RW + ci * CB, CB)
        pltpu.sync_copy(tgt_hbm.at[pl.ds(b0, CB)], tgt_idx)
        r0 = pl.multiple_of(b0 * C, CB * C)
        pltpu.sync_copy(ctx_hbm.at[pl.ds(r0, CB * C)], ctx_idx)
        copies = [pltpu.make_async_copy(in_hbm.at[tgt_idx], v_rows, sem)]
        copies[0].start()
        for j in range(NIDX):
            c = pltpu.make_async_copy(
                out_hbm.at[ctx_idx.at[pl.ds(j * IDX_W, IDX_W)]],
                u_rows.at[pl.ds(j * IDX_W, IDX_W)], sem)
            c.start()
            copies.append(c)
        for c in copies:
            c.wait()

        def row_body(b, carry2):
            v0 = v_rows[b, pl.ds(0, LANES)]
            v1 = v_rows[b, pl.ds(16, LANES)]
            v2 = v_rows[b, pl.ds(32, LANES)]
            v3 = v_rows[b, pl.ds(48, LANES)]
            base = b * C
            for g in range(CPAD // LANES):
                res = jnp.zeros((LANES,), jnp.float32)
                n_in_group = min(LANES, C - g * LANES)
                for t in range(n_in_group):
                    row = base + g * LANES + t
                    acc = u_rows[row, pl.ds(0, LANES)] * v0
                    acc += u_rows[row, pl.ds(16, LANES)] * v1
                    acc += u_rows[row, pl.ds(32, LANES)] * v2
                    acc += u_rows[row, pl.ds(48, LANES)] * v3
                    res = jnp.where(lane == t, jnp.sum(acc), res)
                out_v[b, pl.ds(g * LANES, LANES)] = res
            return carry2

        lax.fori_loop(0, CB, row_body, 0)
        pltpu.sync_copy(out_v, scores_hbm.at[pl.ds(b0, CB)])
        return carry

    lax.fori_loop(0, NCHUNK, chunk_body, 0)


def _scores(target, ctx2, in_table, out_table):
    mesh = plsc.VectorSubcoreMesh(core_axis_name="c", subcore_axis_name="s")
    return pl.kernel(
        _body,
        out_type=jax.ShapeDtypeStruct((B, CPAD), jnp.float32),
        mesh=mesh,
        scratch_types=[
            pltpu.VMEM((CB,), jnp.int32),
            pltpu.VMEM((CB * C,), jnp.int32),
            pltpu.VMEM((CB, EMBED), jnp.float32),
            pltpu.VMEM((CB * C, EMBED), jnp.float32),
            pltpu.VMEM((CB, CPAD), jnp.float32),
            pltpu.SemaphoreType.DMA,
        ],
        compiler_params=pltpu.CompilerParams(
            needs_layout_passes=False, use_tc_tiling_on_sc=False),
    )(target, ctx2, in_table, out_table)


def kernel(target, pos_context, neg_context, in_table, out_table):
    ctx = jnp.concatenate([pos_context, neg_context], axis=1)
    ctx2 = ctx.reshape(B * C).astype(jnp.int32)
    scores = _scores(target.astype(jnp.int32), ctx2, in_table, out_table)
    return scores[:, :P], scores[:, P:C]
